# baseline (speedup 1.0000x reference)
import jax
import jax.numpy as jnp
from jax.experimental import pallas as pl
from jax.experimental.pallas import tpu as pltpu

_ROWS = 2048


def _body(x_ref, c_ref, o_ref):
    x = x_ref[...]
    m = jnp.max(x, axis=-1, keepdims=True)
    e = jnp.exp(x - m)
    s = jnp.sum(e, axis=-1, keepdims=True)
    proj = jnp.dot(e, c_ref[...], preferred_element_type=jnp.float32)
    o_ref[...] = proj / s


def kernel(base_logits, compatibility_matrix):
    b, seq, vocab = base_logits.shape
    diac = compatibility_matrix.shape[1]
    rows = b * seq
    x = base_logits.reshape(rows, vocab)

    out = pl.pallas_call(
        _body,
        grid=(rows // _ROWS,),
        in_specs=[
            pl.BlockSpec((_ROWS, vocab), lambda i: (i, 0)),
            pl.BlockSpec((vocab, diac), lambda i: (0, 0)),
        ],
        out_specs=pl.BlockSpec((_ROWS, diac), lambda i: (i, 0)),
        out_shape=jax.ShapeDtypeStruct((rows, diac), jnp.float32),
        compiler_params=pltpu.CompilerParams(
            dimension_semantics=("arbitrary",),
        ),
    )(x, compatibility_matrix)
    return out.reshape(b, seq, diac)
